# baseline (speedup 1.0000x reference)
import jax
import jax.numpy as jnp
from jax import lax
from jax.experimental import pallas as pl
from jax.experimental.pallas import tpu as pltpu
from jax.experimental.pallas import tpu_sc as plsc

N = 16384
DIMS = (16, 16, 16, 32, 32, 32, 32)
COLS = (0, 16, 32, 48, 80, 112, 144)
TOTAL = 176
V = 100000
VA = 99968

NC, NS = 2, 16
NW = NC * NS
BPW = N // NW
CHUNK = 128
NCH = BPW // CHUNK

TW = (16, 16, 16, 32, 32)
CW16, CW32 = 2048, 1024
NU16 = VA // CW16 + 1
NU32 = VA // CW32 + 1
_STARTS = (0, NU16, 2 * NU16, 3 * NU16, 3 * NU16 + NU32)
NUNITS = 3 * NU16 + 2 * NU32
ROUNDS = -(-NUNITS // NW)
_TOFF = (0, 2048, 4096, 6144, 10240)

_mesh = plsc.VectorSubcoreMesh(core_axis_name="c", subcore_axis_name="s")


def _a_body(in_t, w0, w1, w2, w3, w4, tails, idx_out,
            f0, f1, f2, f3, f4, a16, a32, cbuf, ibuf, iobuf):
    tabs = (w0, w1, w2, w3, w4)
    fouts = (f0, f1, f2, f3, f4)
    wid = lax.axis_index("s") * NC + lax.axis_index("c")
    lane = lax.iota(jnp.int32, 16)

    base = wid * BPW
    pltpu.sync_copy(in_t.at[:, pl.ds(base, BPW)], ibuf)
    for j in range(7):

        def _ib(g, carry, j=j):
            v = ibuf[2 + j, pl.ds(g * 16, 16)]
            iobuf[pl.ds(j * BPW + g * 16, 16)] = v
            return carry
        lax.fori_loop(0, BPW // 16, _ib, 0)
        pltpu.sync_copy(iobuf.at[pl.ds(j * BPW, BPW)],
                        idx_out.at[pl.ds(j * N + base, BPW)])

    @pl.when(wid == NW - 1)
    def _tails():
        for t in range(5):
            w = TW[t]
            pltpu.sync_copy(tails.at[pl.ds(_TOFF[t], 128 * w)],
                            cbuf.at[pl.ds(0, 128 * w)])
            pltpu.sync_copy(cbuf.at[pl.ds(0, 128 * w)],
                            fouts[t].at[pl.ds((V - 128) * w, 128 * w)])

    def _unit16(t, k):
        c0 = jnp.minimum(k * CW16, VA - CW16)
        pltpu.sync_copy(tabs[t].at[:, pl.ds(c0, CW16)], a16)

        def _tr(i, carry):
            r0 = i * 16
            for q in range(16):
                v = plsc.load_gather(
                    a16, [lane, lax.full((16,), 0, jnp.int32) + (r0 + q)])
                cbuf[pl.ds((r0 + q) * 16, 16)] = v
            return carry
        lax.fori_loop(0, CW16 // 16, _tr, 0)
        pltpu.sync_copy(cbuf.at[pl.ds(0, CW16 * 16)],
                        fouts[t].at[pl.ds(c0 * 16, CW16 * 16)])

    def _unit32(t, k):
        c0 = jnp.minimum(k * CW32, VA - CW32)
        pltpu.sync_copy(tabs[t].at[:, pl.ds(c0, CW32)], a32)

        def _tr(i, carry):
            r0 = i * 16
            for q in range(16):
                ci = lax.full((16,), 0, jnp.int32) + (r0 + q)
                vlo = plsc.load_gather(a32, [lane, ci])
                vhi = plsc.load_gather(a32, [lane + 16, ci])
                cbuf[pl.ds((r0 + q) * 32, 16)] = vlo
                cbuf[pl.ds((r0 + q) * 32 + 16, 16)] = vhi
            return carry
        lax.fori_loop(0, CW32 // 16, _tr, 0)
        pltpu.sync_copy(cbuf.at[pl.ds(0, CW32 * 32)],
                        fouts[t].at[pl.ds(c0 * 32, CW32 * 32)])

    for r in range(ROUNDS):
        u = r * NW + wid

        @pl.when(u < NUNITS)
        def _do(u=u):
            branches = [
                lambda u=u, t=t: (_unit16(t, u - _STARTS[t]) if TW[t] == 16
                                  else _unit32(t, u - _STARTS[t]))
                for t in range(5)
            ]
            t_ix = ((u >= _STARTS[1]).astype(jnp.int32)
                    + (u >= _STARTS[2]).astype(jnp.int32)
                    + (u >= _STARTS[3]).astype(jnp.int32)
                    + (u >= _STARTS[4]).astype(jnp.int32))
            lax.switch(t_ix, branches)


_conv = pl.kernel(
    _a_body,
    out_type=(
        jax.ShapeDtypeStruct((7 * N,), jnp.int32),
        jax.ShapeDtypeStruct((V * 16,), jnp.float32),
        jax.ShapeDtypeStruct((V * 16,), jnp.float32),
        jax.ShapeDtypeStruct((V * 16,), jnp.float32),
        jax.ShapeDtypeStruct((V * 32,), jnp.float32),
        jax.ShapeDtypeStruct((V * 32,), jnp.float32),
    ),
    mesh=_mesh,
    compiler_params=pltpu.CompilerParams(use_tc_tiling_on_sc=True,
                                         needs_layout_passes=False),
    scratch_types=[
        pltpu.VMEM((16, CW16), jnp.float32),
        pltpu.VMEM((32, CW32), jnp.float32),
        pltpu.VMEM((CW16 * 16,), jnp.float32),
        pltpu.VMEM((9, BPW), jnp.int32),
        pltpu.VMEM((7 * BPW,), jnp.int32),
    ],
)


def _b_body(idx_hbm, wh, wl, wr, wlon, wlat, out, idx_v, r16, r32,
            gsems, wsem):
    tables = (wh, wl, wr, wlon, wlat, wlon, wlat)
    wid = lax.axis_index("s") * NC + lax.axis_index("c")
    base = wid * BPW

    def rbuf(j, c):
        return r16.at[j, c] if j < 3 else r32.at[j - 3, c]

    gathers = []
    for j in range(7):
        pltpu.sync_copy(idx_hbm.at[j, wid], idx_v.at[j])
        for c in range(NCH):
            gathers.append(
                pltpu.async_copy(tables[j].at[idx_v.at[j, c]], rbuf(j, c),
                                 gsems.at[j]))

    writes = []
    for j in range(7):
        for c in range(NCH):
            gathers[j * NCH + c].wait()
        for c in range(NCH):
            writes.append(
                pltpu.async_copy(
                    rbuf(j, c),
                    out.at[pl.ds(base + c * CHUNK, CHUNK),
                           pl.ds(COLS[j], DIMS[j])],
                    wsem))
    for w in writes:
        w.wait()


_emb = pl.kernel(
    _b_body,
    out_type=jax.ShapeDtypeStruct((N, TOTAL), jnp.float32),
    mesh=_mesh,
    compiler_params=pltpu.CompilerParams(use_tc_tiling_on_sc=False),
    scratch_types=[
        pltpu.VMEM((7, NCH, CHUNK), jnp.int32),
        pltpu.VMEM((3, NCH, CHUNK, 16), jnp.float32),
        pltpu.VMEM((4, NCH, CHUNK, 32), jnp.float32),
        pltpu.SemaphoreType.DMA((7,)),
        pltpu.SemaphoreType.DMA,
    ],
)


def kernel(inputs, W_highway, W_length, W_radian, W_lon, W_lat):
    ws = (W_highway, W_length, W_radian, W_lon, W_lat)
    in_t = jnp.transpose(inputs)
    wts = [jnp.transpose(w) for w in ws]
    tails = jnp.concatenate([w[V - 128:].ravel() for w in ws])
    idxf, f0, f1, f2, f3, f4 = _conv(in_t, *wts, tails)
    return _emb(idxf.reshape(7, NW, NCH, CHUNK),
                f0.reshape(V, 16), f1.reshape(V, 16), f2.reshape(V, 16),
                f3.reshape(V, 32), f4.reshape(V, 32))


# speedup vs baseline: 1.9999x; 1.9999x over previous
import jax
import jax.numpy as jnp
from jax import lax
from jax.experimental import pallas as pl
from jax.experimental.pallas import tpu as pltpu
from jax.experimental.pallas import tpu_sc as plsc

N = 16384
DIMS = (16, 16, 16, 32, 32, 32, 32)
COLS = (0, 16, 32, 48, 80, 112, 144)
TOTAL = 176
V = 100000
VA = 99968

NC, NS = 2, 16
NW = NC * NS
BPW = N // NW
CHUNK = 128
NCH = BPW // CHUNK

TW = (16, 16, 16, 32, 32)
CW16, CW32 = 2048, 1024
NU16 = VA // CW16 + 1
NU32 = VA // CW32 + 1
_STARTS = (0, NU16, 2 * NU16, 3 * NU16, 3 * NU16 + NU32)
NUNITS = 3 * NU16 + 2 * NU32
ROUNDS = -(-NUNITS // NW)
_TOFF = (0, 2048, 4096, 6144, 10240)

_mesh = plsc.VectorSubcoreMesh(core_axis_name="c", subcore_axis_name="s")


def _a_body(in_t, w0, w1, w2, w3, w4, tails, idx_out,
            f0, f1, f2, f3, f4, a16, a32, cbuf, ibuf, iobuf):
    tabs = (w0, w1, w2, w3, w4)
    fouts = (f0, f1, f2, f3, f4)
    wid = lax.axis_index("s") * NC + lax.axis_index("c")
    lane = lax.iota(jnp.int32, 16)

    base = wid * BPW
    pltpu.sync_copy(in_t.at[:, pl.ds(base, BPW)], ibuf)
    for j in range(7):

        def _ib(g, carry, j=j):
            v = ibuf[2 + j, pl.ds(g * 16, 16)]
            iobuf[pl.ds(j * BPW + g * 16, 16)] = v
            return carry
        lax.fori_loop(0, BPW // 16, _ib, 0)
        pltpu.sync_copy(iobuf.at[pl.ds(j * BPW, BPW)],
                        idx_out.at[pl.ds(j * N + base, BPW)])

    @pl.when(wid == NW - 1)
    def _tails():
        for t in range(5):
            w = TW[t]
            pltpu.sync_copy(tails.at[pl.ds(_TOFF[t], 128 * w)],
                            cbuf.at[pl.ds(0, 128 * w)])
            pltpu.sync_copy(cbuf.at[pl.ds(0, 128 * w)],
                            fouts[t].at[pl.ds((V - 128) * w, 128 * w)])

    def _unit16(t, k):
        c0 = jnp.minimum(k * CW16, VA - CW16)
        pltpu.sync_copy(tabs[t].at[:, pl.ds(c0, CW16)], a16)

        def _tr(b, carry):
            cb = b * 16
            for q in range(16):
                perm = (lane + q) & 15
                v = plsc.load_gather(a16, [lane, perm + cb])
                plsc.store_scatter(cbuf, [(perm + cb) * 16 + lane], v)
            return carry
        lax.fori_loop(0, CW16 // 16, _tr, 0)
        pltpu.sync_copy(cbuf.at[pl.ds(0, CW16 * 16)],
                        fouts[t].at[pl.ds(c0 * 16, CW16 * 16)])

    def _unit32(t, k):
        c0 = jnp.minimum(k * CW32, VA - CW32)
        pltpu.sync_copy(tabs[t].at[:, pl.ds(c0, CW32)], a32)

        def _tr(b, carry):
            cb = b * 16
            for q in range(16):
                perm = (lane + q) & 15
                vlo = plsc.load_gather(a32, [lane, perm + cb])
                vhi = plsc.load_gather(a32, [lane + 16, perm + cb])
                plsc.store_scatter(cbuf, [(perm + cb) * 32 + lane], vlo)
                plsc.store_scatter(cbuf, [(perm + cb) * 32 + 16 + lane], vhi)
            return carry
        lax.fori_loop(0, CW32 // 16, _tr, 0)
        pltpu.sync_copy(cbuf.at[pl.ds(0, CW32 * 32)],
                        fouts[t].at[pl.ds(c0 * 32, CW32 * 32)])

    for r in range(ROUNDS):
        u = r * NW + wid

        @pl.when(u < NUNITS)
        def _do(u=u):
            branches = [
                lambda u=u, t=t: (_unit16(t, u - _STARTS[t]) if TW[t] == 16
                                  else _unit32(t, u - _STARTS[t]))
                for t in range(5)
            ]
            t_ix = ((u >= _STARTS[1]).astype(jnp.int32)
                    + (u >= _STARTS[2]).astype(jnp.int32)
                    + (u >= _STARTS[3]).astype(jnp.int32)
                    + (u >= _STARTS[4]).astype(jnp.int32))
            lax.switch(t_ix, branches)


_conv = pl.kernel(
    _a_body,
    out_type=(
        jax.ShapeDtypeStruct((7 * N,), jnp.int32),
        jax.ShapeDtypeStruct((V * 16,), jnp.float32),
        jax.ShapeDtypeStruct((V * 16,), jnp.float32),
        jax.ShapeDtypeStruct((V * 16,), jnp.float32),
        jax.ShapeDtypeStruct((V * 32,), jnp.float32),
        jax.ShapeDtypeStruct((V * 32,), jnp.float32),
    ),
    mesh=_mesh,
    compiler_params=pltpu.CompilerParams(use_tc_tiling_on_sc=True,
                                         needs_layout_passes=False),
    scratch_types=[
        pltpu.VMEM((16, CW16), jnp.float32),
        pltpu.VMEM((32, CW32), jnp.float32),
        pltpu.VMEM((CW16 * 16,), jnp.float32),
        pltpu.VMEM((9, BPW), jnp.int32),
        pltpu.VMEM((7 * BPW,), jnp.int32),
    ],
)


def _b_body(idx_hbm, wh, wl, wr, wlon, wlat, out, idx_v, r16, r32,
            gsems, wsem):
    tables = (wh, wl, wr, wlon, wlat, wlon, wlat)
    wid = lax.axis_index("s") * NC + lax.axis_index("c")
    base = wid * BPW

    def rbuf(j, c):
        return r16.at[j, c] if j < 3 else r32.at[j - 3, c]

    gathers = []
    for j in range(7):
        pltpu.sync_copy(idx_hbm.at[j, wid], idx_v.at[j])
        for c in range(NCH):
            gathers.append(
                pltpu.async_copy(tables[j].at[idx_v.at[j, c]], rbuf(j, c),
                                 gsems.at[j]))

    writes = []
    for j in range(7):
        for c in range(NCH):
            gathers[j * NCH + c].wait()
        for c in range(NCH):
            writes.append(
                pltpu.async_copy(
                    rbuf(j, c),
                    out.at[pl.ds(base + c * CHUNK, CHUNK),
                           pl.ds(COLS[j], DIMS[j])],
                    wsem))
    for w in writes:
        w.wait()


_emb = pl.kernel(
    _b_body,
    out_type=jax.ShapeDtypeStruct((N, TOTAL), jnp.float32),
    mesh=_mesh,
    compiler_params=pltpu.CompilerParams(use_tc_tiling_on_sc=False),
    scratch_types=[
        pltpu.VMEM((7, NCH, CHUNK), jnp.int32),
        pltpu.VMEM((3, NCH, CHUNK, 16), jnp.float32),
        pltpu.VMEM((4, NCH, CHUNK, 32), jnp.float32),
        pltpu.SemaphoreType.DMA((7,)),
        pltpu.SemaphoreType.DMA,
    ],
)


def kernel(inputs, W_highway, W_length, W_radian, W_lon, W_lat):
    ws = (W_highway, W_length, W_radian, W_lon, W_lat)
    in_t = jnp.transpose(inputs)
    wts = [jnp.transpose(w) for w in ws]
    tails = jnp.concatenate([w[V - 128:].ravel() for w in ws])
    idxf, f0, f1, f2, f3, f4 = _conv(in_t, *wts, tails)
    return _emb(idxf.reshape(7, NW, NCH, CHUNK),
                f0.reshape(V, 16), f1.reshape(V, 16), f2.reshape(V, 16),
                f3.reshape(V, 32), f4.reshape(V, 32))


# speedup vs baseline: 2.3785x; 1.1893x over previous
import jax
import jax.numpy as jnp
from jax import lax
from jax.experimental import pallas as pl
from jax.experimental.pallas import tpu as pltpu
from jax.experimental.pallas import tpu_sc as plsc

N = 16384
DIMS = (16, 16, 16, 32, 32, 32, 32)
COLS = (0, 16, 32, 48, 80, 112, 144)
TOTAL = 176
V = 100000
VA = 99968

NC, NS = 2, 16
NW = NC * NS
BPW = N // NW
CHUNK = 128
NCH = BPW // CHUNK

TW = (16, 16, 16, 32, 32)
CW16, CW32 = 1024, 512
NU16 = VA // CW16 + 1
NU32 = VA // CW32 + 1
_STARTS = (0, NU16, 2 * NU16, 3 * NU16, 3 * NU16 + NU32)
NUNITS = 3 * NU16 + 2 * NU32
ROUNDS = -(-NUNITS // NW)
SLOT = CW16 * 16
_TOFF = (0, 2048, 4096, 6144, 10240)

_mesh = plsc.VectorSubcoreMesh(core_axis_name="c", subcore_axis_name="s")


def _a_body(in_t, w0, w1, w2, w3, w4, tails, idx_out,
            f0, f1, f2, f3, f4, a16, a32, cbuf, ibuf, iobuf, isems, osems):
    tabs = (w0, w1, w2, w3, w4)
    fouts = (f0, f1, f2, f3, f4)
    wid = lax.axis_index("s") * NC + lax.axis_index("c")
    lane = lax.iota(jnp.int32, 16)

    base = wid * BPW
    pltpu.sync_copy(in_t.at[:, pl.ds(base, BPW)], ibuf)
    for j in range(7):

        def _ib(g, carry, j=j):
            v = ibuf[2 + j, pl.ds(g * 16, 16)]
            iobuf[pl.ds(j * BPW + g * 16, 16)] = v
            return carry
        lax.fori_loop(0, BPW // 16, _ib, 0)
        pltpu.sync_copy(iobuf.at[pl.ds(j * BPW, BPW)],
                        idx_out.at[pl.ds(j * N + base, BPW)])

    @pl.when(wid == NW - 1)
    def _tails():
        for t in range(5):
            w = TW[t]
            pltpu.sync_copy(tails.at[pl.ds(_TOFF[t], 128 * w)],
                            cbuf.at[pl.ds(0, 128 * w)])
            pltpu.sync_copy(cbuf.at[pl.ds(0, 128 * w)],
                            fouts[t].at[pl.ds((V - 128) * w, 128 * w)])

    def _decode(u):
        t_ix = ((u >= _STARTS[1]).astype(jnp.int32)
                + (u >= _STARTS[2]).astype(jnp.int32)
                + (u >= _STARTS[3]).astype(jnp.int32)
                + (u >= _STARTS[4]).astype(jnp.int32))
        return t_ix

    def _src(t, u):
        if TW[t] == 16:
            c0 = jnp.minimum((u - _STARTS[t]) * CW16, VA - CW16)
            return c0, tabs[t].at[:, pl.ds(c0, CW16)]
        c0 = jnp.minimum((u - _STARTS[t]) * CW32, VA - CW32)
        return c0, tabs[t].at[:, pl.ds(c0, CW32)]

    def _abuf_dst(t, slot):
        if TW[t] == 16:
            return a16.at[pl.ds(slot * 16, 16), :]
        return a32.at[pl.ds(slot * 32, 32), :]

    def _fire_in(r):
        u = r * NW + wid
        slot = r % 2

        @pl.when(u < NUNITS)
        def _():
            def _fire(t):
                pltpu.async_copy(_src(t, u)[1], _abuf_dst(t, slot),
                                 isems.at[slot])
            lax.switch(_decode(u), [lambda t=t: _fire(t) for t in range(5)])

    def _do_unit(r):
        u = r * NW + wid
        slot = r % 2
        cb0 = slot * SLOT

        @pl.when(u < NUNITS)
        def _():
            def _go(t):
                c0, src = _src(t, u)
                pltpu.make_async_copy(src, _abuf_dst(t, slot),
                                      isems.at[slot]).wait()
                if r >= 2:
                    pltpu.make_async_copy(
                        cbuf.at[pl.ds(cb0, SLOT)],
                        fouts[0].at[pl.ds(0, SLOT)],
                        osems.at[slot]).wait()
                if TW[t] == 16:
                    def _tr(b, carry):
                        cbk = b * 16
                        for q in range(16):
                            perm = (lane + q) & 15
                            v = plsc.load_gather(a16, [lane + slot * 16,
                                                       perm + cbk])
                            plsc.store_scatter(
                                cbuf, [cb0 + (perm + cbk) * 16 + lane], v)
                        return carry
                    lax.fori_loop(0, CW16 // 16, _tr, 0)
                    pltpu.async_copy(cbuf.at[pl.ds(cb0, SLOT)],
                                     fouts[t].at[pl.ds(c0 * 16, SLOT)],
                                     osems.at[slot])
                else:
                    def _tr(b, carry):
                        cbk = b * 16
                        for q in range(16):
                            perm = (lane + q) & 15
                            vlo = plsc.load_gather(a32, [lane + slot * 32,
                                                         perm + cbk])
                            vhi = plsc.load_gather(a32, [lane + slot * 32 + 16,
                                                         perm + cbk])
                            plsc.store_scatter(
                                cbuf, [cb0 + (perm + cbk) * 32 + lane], vlo)
                            plsc.store_scatter(
                                cbuf, [cb0 + (perm + cbk) * 32 + 16 + lane],
                                vhi)
                        return carry
                    lax.fori_loop(0, CW32 // 16, _tr, 0)
                    pltpu.async_copy(cbuf.at[pl.ds(cb0, SLOT)],
                                     fouts[t].at[pl.ds(c0 * 32, SLOT)],
                                     osems.at[slot])

            lax.switch(_decode(u), [lambda t=t: _go(t) for t in range(5)])

    _fire_in(0)
    for r in range(ROUNDS):
        _fire_in(r + 1)
        _do_unit(r)
    for slot in (0, 1):
        pltpu.make_async_copy(cbuf.at[pl.ds(slot * SLOT, SLOT)],
                              fouts[0].at[pl.ds(0, SLOT)],
                              osems.at[slot]).wait()


_conv = pl.kernel(
    _a_body,
    out_type=(
        jax.ShapeDtypeStruct((7 * N,), jnp.int32),
        jax.ShapeDtypeStruct((V * 16,), jnp.float32),
        jax.ShapeDtypeStruct((V * 16,), jnp.float32),
        jax.ShapeDtypeStruct((V * 16,), jnp.float32),
        jax.ShapeDtypeStruct((V * 32,), jnp.float32),
        jax.ShapeDtypeStruct((V * 32,), jnp.float32),
    ),
    mesh=_mesh,
    compiler_params=pltpu.CompilerParams(use_tc_tiling_on_sc=True,
                                         needs_layout_passes=False),
    scratch_types=[
        pltpu.VMEM((32, CW16), jnp.float32),
        pltpu.VMEM((64, CW32), jnp.float32),
        pltpu.VMEM((2 * SLOT,), jnp.float32),
        pltpu.VMEM((9, BPW), jnp.int32),
        pltpu.VMEM((7 * BPW,), jnp.int32),
        pltpu.SemaphoreType.DMA((2,)),
        pltpu.SemaphoreType.DMA((2,)),
    ],
)


def _b_body(idx_hbm, wh, wl, wr, wlon, wlat, out, idx_v, r16, r32,
            gsems, wsem):
    tables = (wh, wl, wr, wlon, wlat, wlon, wlat)
    wid = lax.axis_index("s") * NC + lax.axis_index("c")
    base = wid * BPW

    def rbuf(j, c):
        return r16.at[j, c] if j < 3 else r32.at[j - 3, c]

    gathers = []
    for j in range(7):
        pltpu.sync_copy(idx_hbm.at[j, wid], idx_v.at[j])
        for c in range(NCH):
            gathers.append(
                pltpu.async_copy(tables[j].at[idx_v.at[j, c]], rbuf(j, c),
                                 gsems.at[j]))

    writes = []
    for j in range(7):
        for c in range(NCH):
            gathers[j * NCH + c].wait()
        for c in range(NCH):
            writes.append(
                pltpu.async_copy(
                    rbuf(j, c),
                    out.at[pl.ds(base + c * CHUNK, CHUNK),
                           pl.ds(COLS[j], DIMS[j])],
                    wsem))
    for w in writes:
        w.wait()


_emb = pl.kernel(
    _b_body,
    out_type=jax.ShapeDtypeStruct((N, TOTAL), jnp.float32),
    mesh=_mesh,
    compiler_params=pltpu.CompilerParams(use_tc_tiling_on_sc=False),
    scratch_types=[
        pltpu.VMEM((7, NCH, CHUNK), jnp.int32),
        pltpu.VMEM((3, NCH, CHUNK, 16), jnp.float32),
        pltpu.VMEM((4, NCH, CHUNK, 32), jnp.float32),
        pltpu.SemaphoreType.DMA((7,)),
        pltpu.SemaphoreType.DMA,
    ],
)


def kernel(inputs, W_highway, W_length, W_radian, W_lon, W_lat):
    ws = (W_highway, W_length, W_radian, W_lon, W_lat)
    in_t = jnp.transpose(inputs)
    wts = [jnp.transpose(w) for w in ws]
    tails = jnp.concatenate([w[V - 128:].ravel() for w in ws])
    idxf, f0, f1, f2, f3, f4 = _conv(in_t, *wts, tails)
    return _emb(idxf.reshape(7, NW, NCH, CHUNK),
                f0.reshape(V, 16), f1.reshape(V, 16), f2.reshape(V, 16),
                f3.reshape(V, 32), f4.reshape(V, 32))
